# baseline (speedup 1.0000x reference)
import jax
import jax.numpy as jnp
from jax import lax
from jax.experimental import pallas as pl
from jax.experimental.pallas import tpu as pltpu
from jax.experimental.pallas import tpu_sc as plsc

_INFO = plsc.get_sparse_core_info()
_NC = _INFO.num_cores
_NS = _INFO.num_subcores
_NW = _NC * _NS
_L = _INFO.num_lanes

_B, _D1, _D2, _D3 = 4, 160, 160, 160
_PLANES_PER_W = (_B * _D1) // _NW
_ROWS = 40
_CPP = _D2 // _ROWS
_NCHUNK = _PLANES_PER_W * _CPP
_NBUF = 4
_C0 = 128
_C1 = _D3 - _C0
_LUT_SIZE = 61


def _body(labels_hbm, lut_hbm, out_hbm, lut_v, in0, in1, out0, out1,
          in_sems0, in_sems1, out_sems0, out_sems1):
    wid = lax.axis_index("s") * _NC + lax.axis_index("c")
    d0 = wid // (_NW // _B)
    d1_base = (wid % (_NW // _B)) * _PLANES_PER_W

    pltpu.sync_copy(lut_hbm, lut_v)

    def loc(g):
        return d1_base + (g >> 2), (g & (_CPP - 1)) * _ROWS

    def in_copies(g, b):
        d1, r0 = loc(g)
        return (
            pltpu.make_async_copy(
                labels_hbm.at[d0, d1, pl.ds(r0, _ROWS), pl.ds(0, _C0)],
                in0[b], in_sems0[b]),
            pltpu.make_async_copy(
                labels_hbm.at[d0, d1, pl.ds(r0, _ROWS), pl.ds(_C0, _C1)],
                in1[b], in_sems1[b]),
        )

    def out_copies(g, b):
        d1, r0 = loc(g)
        return (
            pltpu.make_async_copy(
                out0[b], out_hbm.at[d0, d1, pl.ds(r0, _ROWS), pl.ds(0, _C0)],
                out_sems0[b]),
            pltpu.make_async_copy(
                out1[b],
                out_hbm.at[d0, d1, pl.ds(r0, _ROWS), pl.ds(_C0, _C1)],
                out_sems1[b]),
        )

    for b in range(_NBUF):
        for cp in in_copies(b, b):
            cp.start()

    def chunk_group(gg, _):
        for b in range(_NBUF):
            g = gg * _NBUF + b
            for cp in in_copies(g, b):
                cp.wait()

            @pl.when(g >= _NBUF)
            def _(b=b, g=g):
                for cp in out_copies(g - _NBUF, b):
                    cp.wait()

            def step(r, _, b=b):
                idxs = ([in0[b][r, pl.ds(c * _L, _L)]
                         for c in range(_C0 // _L)]
                        + [in1[b][r, pl.ds(c * _L, _L)]
                           for c in range(_C1 // _L)])
                vals = [plsc.load_gather(lut_v, [ix]) for ix in idxs]
                for c in range(_C0 // _L):
                    out0[b][r, pl.ds(c * _L, _L)] = vals[c]
                for c in range(_C1 // _L):
                    out1[b][r, pl.ds(c * _L, _L)] = vals[_C0 // _L + c]
                return 0

            lax.fori_loop(0, _ROWS, step, 0, unroll=2)

            for cp in out_copies(g, b):
                cp.start()

            @pl.when(g + _NBUF < _NCHUNK)
            def _(b=b, g=g):
                for cp in in_copies(g + _NBUF, b):
                    cp.start()

        return 0

    lax.fori_loop(0, _NCHUNK // _NBUF, chunk_group, 0)

    for g in range(_NCHUNK - _NBUF, _NCHUNK):
        for cp in out_copies(g, g % _NBUF):
            cp.wait()


def kernel(labels, lut):
    run = pl.kernel(
        _body,
        out_type=jax.ShapeDtypeStruct((_B, _D1, _D2, _D3), jnp.int32),
        mesh=plsc.VectorSubcoreMesh(core_axis_name="c", subcore_axis_name="s"),
        scratch_types=[
            pltpu.VMEM((_LUT_SIZE,), jnp.int32),
            [pltpu.VMEM((_ROWS, _C0), jnp.int32) for _ in range(_NBUF)],
            [pltpu.VMEM((_ROWS, _C1), jnp.int32) for _ in range(_NBUF)],
            [pltpu.VMEM((_ROWS, _C0), jnp.int32) for _ in range(_NBUF)],
            [pltpu.VMEM((_ROWS, _C1), jnp.int32) for _ in range(_NBUF)],
            [pltpu.SemaphoreType.DMA for _ in range(_NBUF)],
            [pltpu.SemaphoreType.DMA for _ in range(_NBUF)],
            [pltpu.SemaphoreType.DMA for _ in range(_NBUF)],
            [pltpu.SemaphoreType.DMA for _ in range(_NBUF)],
        ],
        compiler_params=pltpu.CompilerParams(needs_layout_passes=False),
    )
    return run(labels.astype(jnp.int32), lut.astype(jnp.int32))


# speedup vs baseline: 1.0125x; 1.0125x over previous
import jax
import jax.numpy as jnp
from jax import lax
from jax.experimental import pallas as pl
from jax.experimental.pallas import tpu as pltpu
from jax.experimental.pallas import tpu_sc as plsc

_INFO = plsc.get_sparse_core_info()
_NC = _INFO.num_cores
_NS = _INFO.num_subcores
_NW = _NC * _NS
_L = _INFO.num_lanes

_B, _D1, _D2, _D3 = 4, 160, 160, 160
_PLANES_PER_W = (_B * _D1) // _NW
_ROWS = 40
_CPP = _D2 // _ROWS
_NCHUNK = _PLANES_PER_W * _CPP
_NBUF = 4
_LUT_SIZE = 61


def _body(labels_hbm, lut_hbm, out_hbm, lut_v, in_bufs, out_bufs,
          in_sems, out_sems):
    wid = lax.axis_index("s") * _NC + lax.axis_index("c")
    d0 = wid // (_NW // _B)
    d1_base = (wid % (_NW // _B)) * _PLANES_PER_W

    pltpu.sync_copy(lut_hbm, lut_v)

    def in_copy(g, b):
        d1 = d1_base + (g >> 2)
        r0 = (g & (_CPP - 1)) * _ROWS
        return pltpu.make_async_copy(
            labels_hbm.at[d0, d1, pl.ds(r0, _ROWS), :], in_bufs[b],
            in_sems[b])

    def out_copy(g, b):
        d1 = d1_base + (g >> 2)
        r0 = (g & (_CPP - 1)) * _ROWS
        return pltpu.make_async_copy(
            out_bufs[b], out_hbm.at[d0, d1, pl.ds(r0, _ROWS), :],
            out_sems[b])

    for b in range(_NBUF):
        in_copy(b, b).start()

    def chunk_group(gg, _):
        for b in range(_NBUF):
            g = gg * _NBUF + b
            in_copy(g, b).wait()

            @pl.when(g >= _NBUF)
            def _(b=b, g=g):
                out_copy(g - _NBUF, b).wait()

            def step(r, _, b=b):
                idxs = [in_bufs[b][r, pl.ds(c * _L, _L)]
                        for c in range(_D3 // _L)]
                vals = [plsc.load_gather(lut_v, [ix]) for ix in idxs]
                for c in range(_D3 // _L):
                    out_bufs[b][r, pl.ds(c * _L, _L)] = vals[c]
                return 0

            lax.fori_loop(0, _ROWS, step, 0, unroll=2)

            out_copy(g, b).start()

            @pl.when(g + _NBUF < _NCHUNK)
            def _(b=b, g=g):
                in_copy(g + _NBUF, b).start()

        return 0

    lax.fori_loop(0, _NCHUNK // _NBUF, chunk_group, 0)

    for g in range(_NCHUNK - _NBUF, _NCHUNK):
        out_copy(g, g % _NBUF).wait()


def kernel(labels, lut):
    run = pl.kernel(
        _body,
        out_type=jax.ShapeDtypeStruct((_B, _D1, _D2, _D3), jnp.int32),
        mesh=plsc.VectorSubcoreMesh(core_axis_name="c", subcore_axis_name="s"),
        scratch_types=[
            pltpu.VMEM((_LUT_SIZE,), jnp.int32),
            [pltpu.VMEM((_ROWS, _D3), jnp.int32) for _ in range(_NBUF)],
            [pltpu.VMEM((_ROWS, _D3), jnp.int32) for _ in range(_NBUF)],
            [pltpu.SemaphoreType.DMA for _ in range(_NBUF)],
            [pltpu.SemaphoreType.DMA for _ in range(_NBUF)],
        ],
        compiler_params=pltpu.CompilerParams(needs_layout_passes=False),
    )
    return run(labels.astype(jnp.int32), lut.astype(jnp.int32))
